# baseline (speedup 1.0000x reference)
import functools

import jax
import jax.numpy as jnp
from jax import lax
from jax.experimental import pallas as pl
from jax.experimental.pallas import tpu as pltpu
from jax.experimental.pallas import tpu_sc as plsc
from jax.experimental.layout import Format, Layout

DIM = 64
NC, NS = 2, 16
NW = NC * NS
CHUNK = 256
GRP = 16


def _sc_cosine(table, center_idx, context_idx):
    batch = center_idx.shape[0]
    bpw = batch // NW
    nchunks = bpw // CHUNK
    mesh = plsc.VectorSubcoreMesh(core_axis_name="c", subcore_axis_name="s")

    @functools.partial(
        pl.kernel,
        mesh=mesh,
        compiler_params=pltpu.CompilerParams(use_tc_tiling_on_sc=False,
                                             needs_layout_passes=False),
        out_type=jax.ShapeDtypeStruct((batch,), jnp.float32),
        scratch_types=[
            pltpu.VMEM((bpw,), jnp.int32),
            pltpu.VMEM((bpw,), jnp.int32),
            pltpu.VMEM((CHUNK, DIM), jnp.float32),
            pltpu.VMEM((CHUNK, DIM), jnp.float32),
            pltpu.VMEM((bpw,), jnp.float32),
            pltpu.SemaphoreType.DMA,
            pltpu.SemaphoreType.DMA,
        ],
    )
    def k(table_hbm, cen_hbm, ctx_hbm, out_hbm,
          rcen_v, rctx_v, dstc_v, dstx_v, out_v, sem_c, sem_x):
        wid = lax.axis_index("s") * NC + lax.axis_index("c")
        base = wid * bpw
        pltpu.sync_copy(cen_hbm.at[pl.ds(base, bpw)], rcen_v)
        pltpu.sync_copy(ctx_hbm.at[pl.ds(base, bpw)], rctx_v)

        for chunk in range(nchunks):
            cbase = chunk * CHUNK
            cp_c = pltpu.async_copy(
                table_hbm.at[rcen_v.at[pl.ds(cbase, CHUNK)]], dstc_v, sem_c)
            cp_x = pltpu.async_copy(
                table_hbm.at[rctx_v.at[pl.ds(cbase, CHUNK)]], dstx_v, sem_x)
            cp_c.wait()
            cp_x.wait()

            @pl.loop(0, CHUNK // GRP)
            def _compute(g):
                lane = lax.iota(jnp.int32, GRP)
                rows = g * GRP + lane
                dot = jnp.zeros((GRP,), jnp.float32)
                cc = jnp.zeros((GRP,), jnp.float32)
                xx = jnp.zeros((GRP,), jnp.float32)
                for c in range(DIM):
                    rot = (jnp.full((GRP,), c, jnp.int32) + lane) & (DIM - 1)
                    cv = plsc.load_gather(dstc_v, [rows, rot])
                    xv = plsc.load_gather(dstx_v, [rows, rot])
                    dot = dot + cv * xv
                    cc = cc + cv * cv
                    xx = xx + xv * xv
                y = cc * xx
                iy = plsc.bitcast(y, jnp.int32)
                iz = jnp.int32(0x5F3759DF) - lax.shift_right_logical(iy, 1)
                z = plsc.bitcast(iz, jnp.float32)
                for _ in range(3):
                    z = z * (1.5 - 0.5 * y * z * z)
                denom = y * z
                out_v[pl.ds(cbase + g * GRP, GRP)] = dot / (denom + 1e-8)

        pltpu.sync_copy(out_v, out_hbm.at[pl.ds(base, bpw)])

    return k(table, center_idx, context_idx)


@jax.jit
def kernel(center_idx, context_idx, table):
    sharding = jax.sharding.SingleDeviceSharding(jax.devices()[0])
    table_lin = jax.device_put(
        table,
        Format(Layout(major_to_minor=(0, 1), tiling=((16,),)), sharding))
    return _sc_cosine(table_lin,
                      center_idx.astype(jnp.int32),
                      context_idx.astype(jnp.int32))
